# baseline (speedup 1.0000x reference)
import jax
import jax.numpy as jnp
from jax import lax
from jax.experimental import pallas as pl
from jax.experimental.pallas import tpu as pltpu, tpu_sc as plsc

MAX_D = 10
D = 128
NC, NS, L = 2, 16, 16
NW = NC * NS
CHUNK = 128


def _make_kernel(n_pad):
    chunks_per_w = n_pad // (NW * CHUNK)
    rows_per_w = chunks_per_w * CHUNK
    mesh = plsc.VectorSubcoreMesh(core_axis_name="c", subcore_axis_name="s")

    def body(idx_hbm, table_hbm, out_hbm, idx_v, rows_v, semg, semw0, semw1):
        wid = lax.axis_index("s") * NC + lax.axis_index("c")
        pltpu.sync_copy(idx_hbm.at[pl.ds(wid * rows_per_w, rows_per_w)], idx_v)

        for i in range(rows_per_w // L):
            x = idx_v[pl.ds(i * L, L)]
            c = jnp.where(x > 1000, MAX_D + 1, jnp.minimum(x, MAX_D))
            idx_v[pl.ds(i * L, L)] = c

        def idx_at(j):
            return idx_v.at[pl.ds(j * CHUNK, CHUNK)]

        base = wid * rows_per_w
        semw = (semw0, semw1)
        pltpu.async_copy(table_hbm.at[idx_at(0)], rows_v.at[0], semg)
        for j in range(chunks_per_w):
            b = j % 2
            pltpu.make_async_copy(table_hbm.at[idx_at(j)], rows_v.at[b], semg).wait()
            pltpu.async_copy(
                rows_v.at[b], out_hbm.at[pl.ds(base + j * CHUNK, CHUNK)], semw[b]
            )
            if j + 1 < chunks_per_w:
                nb = (j + 1) % 2
                if j >= 1:
                    pltpu.make_async_copy(
                        rows_v.at[nb],
                        out_hbm.at[pl.ds(base + (j - 1) * CHUNK, CHUNK)],
                        semw[nb],
                    ).wait()
                pltpu.async_copy(table_hbm.at[idx_at(j + 1)], rows_v.at[nb], semg)
        last = chunks_per_w - 1
        if chunks_per_w >= 2:
            pltpu.make_async_copy(
                rows_v.at[(last - 1) % 2],
                out_hbm.at[pl.ds(base + (last - 1) * CHUNK, CHUNK)],
                semw[(last - 1) % 2],
            ).wait()
        pltpu.make_async_copy(
            rows_v.at[last % 2],
            out_hbm.at[pl.ds(base + last * CHUNK, CHUNK)],
            semw[last % 2],
        ).wait()

    return pl.kernel(
        body,
        out_type=jax.ShapeDtypeStruct((n_pad, D), jnp.float32),
        mesh=mesh,
        scratch_types=[
            pltpu.VMEM((rows_per_w,), jnp.int32),
            pltpu.VMEM((2, CHUNK, D), jnp.float32),
            pltpu.SemaphoreType.DMA,
            pltpu.SemaphoreType.DMA,
            pltpu.SemaphoreType.DMA,
        ],
    )


@jax.jit
def kernel(data, embed_weight):
    n = data.shape[0]
    n_pad = -(-n // (NW * CHUNK)) * (NW * CHUNK)
    idx = jnp.reshape(data, (-1,)).astype(jnp.int32)
    idx = jnp.pad(idx, (0, n_pad - n))
    out = _make_kernel(n_pad)(idx, embed_weight)
    return out[:n]


# speedup vs baseline: 3.9383x; 3.9383x over previous
import jax
import jax.numpy as jnp
from jax import lax
from jax.experimental import pallas as pl
from jax.experimental.pallas import tpu as pltpu, tpu_sc as plsc

MAX_D = 10
D = 128
NC, NS, L = 2, 16, 16
NW = NC * NS
R = 32


def _make_kernel(n_pad, n_rows_table):
    rows_per_w = n_pad // NW
    nchunks = rows_per_w // R
    npairs = nchunks // 2
    mesh = plsc.VectorSubcoreMesh(core_axis_name="c", subcore_axis_name="s")

    def body(idx_hbm, table_hbm, out_hbm, idx_v, tab_v, outb0, outb1, semw0, semw1):
        iota = lax.iota(jnp.int32, L)
        wid = lax.axis_index("s") * NC + lax.axis_index("c")
        pltpu.sync_copy(table_hbm, tab_v)
        pltpu.sync_copy(idx_hbm.at[pl.ds(wid * rows_per_w, rows_per_w)], idx_v)

        outb = (outb0, outb1)
        semw = (semw0, semw1)
        wbase = wid * rows_per_w

        def compute_chunk(k, buf):
            for jg in range(R // L):
                x = plsc.load_gather(idx_v, [iota + (k * R + jg * L)])
                g = jnp.where(x > 1000, MAX_D + 1, jnp.minimum(x, MAX_D)) * D
                s = (iota + jg * L) * D
                for c in range(D):
                    vals = plsc.load_gather(tab_v, [g + c])
                    plsc.store_scatter(outb[buf], [s + c], vals)

        def write_chunk(k, buf):
            off = pl.multiple_of((wbase + k * R) * D, R * D)
            pltpu.async_copy(outb[buf], out_hbm.at[pl.ds(off, R * D)], semw[buf])

        def wait_chunk(buf):
            pltpu.make_async_copy(
                outb[buf], out_hbm.at[pl.ds(0, R * D)], semw[buf]
            ).wait()

        compute_chunk(0, 0)
        write_chunk(0, 0)
        compute_chunk(1, 1)
        write_chunk(1, 1)

        def pair(p, carry):
            for b in range(2):
                k = p * 2 + b
                wait_chunk(b)
                compute_chunk(k, b)
                write_chunk(k, b)
            return carry

        lax.fori_loop(1, npairs, pair, 0)
        wait_chunk(0)
        wait_chunk(1)

    return pl.kernel(
        body,
        out_type=jax.ShapeDtypeStruct((n_pad * D,), jnp.float32),
        mesh=mesh,
        compiler_params=pltpu.CompilerParams(needs_layout_passes=False),
        scratch_types=[
            pltpu.VMEM((rows_per_w,), jnp.int32),
            pltpu.VMEM((n_rows_table * D,), jnp.float32),
            pltpu.VMEM((R * D,), jnp.float32),
            pltpu.VMEM((R * D,), jnp.float32),
            pltpu.SemaphoreType.DMA,
            pltpu.SemaphoreType.DMA,
        ],
    )


@jax.jit
def kernel(data, embed_weight):
    n = data.shape[0]
    grain = NW * R * 2
    n_pad = -(-n // grain) * grain
    idx = jnp.reshape(data, (-1,)).astype(jnp.int32)
    idx = jnp.pad(idx, (0, n_pad - n))
    out = _make_kernel(n_pad, embed_weight.shape[0])(
        idx, jnp.reshape(embed_weight, (-1,))
    )
    return jnp.reshape(out, (n_pad, D))[:n]


# speedup vs baseline: 6.2268x; 1.5811x over previous
import jax
import jax.numpy as jnp
from jax import lax
from jax.experimental import pallas as pl
from jax.experimental.pallas import tpu as pltpu, tpu_sc as plsc

MAX_D = 10
D = 128
DP = D + 1
NC, NS, L = 2, 16, 16
NW = NC * NS
R = 32


def _make_kernel(n_pad):
    rows_per_w = n_pad // NW
    nchunks = rows_per_w // R
    npairs = nchunks // 2
    mesh = plsc.VectorSubcoreMesh(core_axis_name="c", subcore_axis_name="s")

    def body(idx_hbm, t16_hbm, out_hbm, idx_v, tab_v, outb0, outb1, semw0, semw1):
        iota = lax.iota(jnp.int32, L)
        wid = lax.axis_index("s") * NC + lax.axis_index("c")
        pltpu.sync_copy(t16_hbm, tab_v)
        pltpu.sync_copy(idx_hbm.at[pl.ds(wid * rows_per_w, rows_per_w)], idx_v)

        outb = (outb0, outb1)
        semw = (semw0, semw1)
        wbase = wid * rows_per_w

        def compute_chunk(k, buf):
            for jg in range(R // L):
                x = plsc.load_gather(idx_v, [iota + (k * R + jg * L)])
                row = jnp.where(x > 1000, MAX_D + 1, jnp.minimum(x, MAX_D))
                g0 = row * (D * L) + iota
                rows = iota + jg * L
                for c in range(D):
                    vals = plsc.load_gather(tab_v, [g0 + c * L])
                    plsc.store_scatter(
                        outb[buf], [rows, jnp.full((L,), c, jnp.int32)], vals
                    )

        def write_chunk(k, buf):
            off = pl.multiple_of((wbase + k * R), R)
            pltpu.async_copy(
                outb[buf].at[:, pl.ds(0, D)],
                out_hbm.at[pl.ds(off, R)],
                semw[buf],
            )

        def wait_chunk(buf):
            pltpu.make_async_copy(
                outb[buf].at[:, pl.ds(0, D)], out_hbm.at[pl.ds(0, R)], semw[buf]
            ).wait()

        def pair(p, carry):
            for b in range(2):
                k = p * 2 + b

                @pl.when(p > 0)
                def _():
                    wait_chunk(b)

                compute_chunk(k, b)
                write_chunk(k, b)
            return carry

        lax.fori_loop(0, npairs, pair, 0)
        wait_chunk(0)
        wait_chunk(1)

    return pl.kernel(
        body,
        out_type=jax.ShapeDtypeStruct((n_pad, D), jnp.float32),
        mesh=mesh,
        compiler_params=pltpu.CompilerParams(needs_layout_passes=False),
        scratch_types=[
            pltpu.VMEM((rows_per_w,), jnp.int32),
            pltpu.VMEM(((MAX_D + 2) * D * L,), jnp.float32),
            pltpu.VMEM((R, DP), jnp.float32),
            pltpu.VMEM((R, DP), jnp.float32),
            pltpu.SemaphoreType.DMA,
            pltpu.SemaphoreType.DMA,
        ],
    )


@jax.jit
def kernel(data, embed_weight):
    n = data.shape[0]
    grain = NW * R * 2
    n_pad = -(-n // grain) * grain
    idx = jnp.reshape(data, (-1,)).astype(jnp.int32)
    idx = jnp.pad(idx, (0, n_pad - n))
    t16 = jnp.broadcast_to(
        jnp.reshape(embed_weight, (-1, 1)), (embed_weight.size, L)
    ).reshape(-1)
    out = _make_kernel(n_pad)(idx, t16)
    return out[:n]


# speedup vs baseline: 9.9165x; 1.5925x over previous
import jax
import jax.numpy as jnp
from jax import lax
from jax.experimental import pallas as pl
from jax.experimental.pallas import tpu as pltpu, tpu_sc as plsc

MAX_D = 10
D = 128
DP = D + 1
NC, NS, L = 2, 16, 16
NW = NC * NS
R = 32


def _make_kernel(n_pad):
    rows_per_w = n_pad // NW
    nchunks = rows_per_w // R
    npairs = nchunks // 2
    mesh = plsc.VectorSubcoreMesh(core_axis_name="c", subcore_axis_name="s")

    def body(idx_hbm, t16_hbm, out_hbm, idx_v, tab_v, outb0, outb1, semw0, semw1):
        iota = lax.iota(jnp.int32, L)
        wid = lax.axis_index("s") * NC + lax.axis_index("c")
        pltpu.sync_copy(t16_hbm, tab_v)
        pltpu.sync_copy(idx_hbm.at[pl.ds(wid * rows_per_w, rows_per_w)], idx_v)

        outb = (outb0, outb1)
        semw = (semw0, semw1)
        wbase = wid * rows_per_w

        def compute_chunk(k, buf):
            for jg in range(R // L):
                x = plsc.load_gather(idx_v, [iota + (k * R + jg * L)])
                row = jnp.where(x > 1000, MAX_D + 1, jnp.minimum(x, MAX_D))
                g0 = row * (D * L) + iota
                zero = lax.shift_right_logical(x, 31)
                s0 = (iota + jg * L) * DP + zero
                for cc in range(D // 2):
                    va = plsc.load_gather(tab_v, [g0 + cc * L])
                    vb = plsc.load_gather(tab_v, [g0 + (cc + D // 2) * L])
                    plsc.store_scatter(outb[buf], [zero, s0 + cc], va)
                    plsc.store_scatter(outb[buf], [zero, s0 + (cc + D // 2)], vb)

        def write_chunk(k, buf):
            off = pl.multiple_of((wbase + k * R), R)
            pltpu.async_copy(
                outb[buf].at[:, pl.ds(0, D)],
                out_hbm.at[pl.ds(off, R)],
                semw[buf],
            )

        def wait_chunk(buf):
            pltpu.make_async_copy(
                outb[buf].at[:, pl.ds(0, D)], out_hbm.at[pl.ds(0, R)], semw[buf]
            ).wait()

        def pair(p, carry):
            for b in range(2):
                k = p * 2 + b

                @pl.when(p > 0)
                def _():
                    wait_chunk(b)

                compute_chunk(k, b)
                write_chunk(k, b)
            return carry

        lax.fori_loop(0, npairs, pair, 0)
        wait_chunk(0)
        wait_chunk(1)

    return pl.kernel(
        body,
        out_type=jax.ShapeDtypeStruct((n_pad, D), jnp.float32),
        mesh=mesh,
        compiler_params=pltpu.CompilerParams(needs_layout_passes=False),
        scratch_types=[
            pltpu.VMEM((rows_per_w,), jnp.int32),
            pltpu.VMEM(((MAX_D + 2) * D * L,), jnp.float32),
            pltpu.VMEM((R, DP), jnp.float32),
            pltpu.VMEM((R, DP), jnp.float32),
            pltpu.SemaphoreType.DMA,
            pltpu.SemaphoreType.DMA,
        ],
    )


@jax.jit
def kernel(data, embed_weight):
    n = data.shape[0]
    grain = NW * R * 2
    n_pad = -(-n // grain) * grain
    idx = jnp.reshape(data, (-1,)).astype(jnp.int32)
    idx = jnp.pad(idx, (0, n_pad - n))
    t16 = jnp.broadcast_to(
        jnp.reshape(embed_weight, (-1, 1)), (embed_weight.size, L)
    ).reshape(-1)
    out = _make_kernel(n_pad)(idx, t16)
    return out[:n]
